# baseline (speedup 1.0000x reference)
import jax
import jax.numpy as jnp
from jax.experimental import pallas as pl
from jax.experimental.pallas import tpu as pltpu


def _linear_kernel(x_ref, w_ref, b_ref, o_ref, wb_ref):
    i = pl.program_id(0)
    kk = pl.program_id(1)

    @pl.when(i == 0)
    def _cast_w():
        wb_ref[kk] = w_ref[...].astype(jnp.bfloat16)

    xb = x_ref[...].astype(jnp.bfloat16)
    acc = jnp.dot(xb, wb_ref[kk], preferred_element_type=jnp.float32)

    @pl.when(kk == 0)
    def _init():
        o_ref[...] = acc + b_ref[...]

    @pl.when(kk != 0)
    def _accum():
        o_ref[...] += acc


def kernel(x, w, b):
    m, k = x.shape
    n = w.shape[1]
    tm = 512
    nk = 2
    tk = k // nk

    grid = (m // tm, nk)
    cost = pl.CostEstimate(
        flops=2 * m * k * n,
        transcendentals=0,
        bytes_accessed=4 * m * k + 4 * k * n + 4 * n + 4 * m * n,
    )
    return pl.pallas_call(
        _linear_kernel,
        out_shape=jax.ShapeDtypeStruct((m, n), jnp.float32),
        grid=grid,
        in_specs=[
            pl.BlockSpec((tm, tk), lambda i, kk: (i, kk)),
            pl.BlockSpec((tk, n), lambda i, kk: (kk, 0)),
            pl.BlockSpec((1, n), lambda i, kk: (0, 0)),
        ],
        out_specs=pl.BlockSpec((tm, n), lambda i, kk: (i, 0)),
        scratch_shapes=[pltpu.VMEM((nk, tk, n), jnp.bfloat16)],
        compiler_params=pltpu.CompilerParams(
            dimension_semantics=("arbitrary", "arbitrary"),
            vmem_limit_bytes=60 << 20,
        ),
        cost_estimate=cost,
    )(x, w, b)


# speedup vs baseline: 1.6399x; 1.6399x over previous
import jax
import jax.numpy as jnp
from jax.experimental import pallas as pl
from jax.experimental.pallas import tpu as pltpu

_TM = 512


def _linear_kernel(x_hbm, w_ref, b_ref, o_hbm,
                   wb_ref, x_buf, o_buf, in_sem, out_sem):
    n_steps = x_hbm.shape[0] // _TM

    def dma_in(slot, step):
        pltpu.make_async_copy(
            x_hbm.at[pl.ds(step * _TM, _TM)], x_buf.at[slot], in_sem.at[slot]
        ).start()

    def wait_in(slot):
        pltpu.make_async_copy(
            x_hbm.at[pl.ds(0, _TM)], x_buf.at[slot], in_sem.at[slot]
        ).wait()

    def dma_out(slot, step):
        pltpu.make_async_copy(
            o_buf.at[slot], o_hbm.at[pl.ds(step * _TM, _TM)], out_sem.at[slot]
        ).start()

    def wait_out(slot):
        pltpu.make_async_copy(
            o_buf.at[slot], o_hbm.at[pl.ds(0, _TM)], out_sem.at[slot]
        ).wait()

    dma_in(0, 0)
    wb_ref[...] = w_ref[...].astype(jnp.bfloat16)

    def body(step, _):
        cur = jax.lax.rem(step, 2)
        nxt = jax.lax.rem(step + 1, 2)

        @pl.when(step + 1 < n_steps)
        def _prefetch():
            dma_in(nxt, step + 1)

        wait_in(cur)

        @pl.when(step >= 2)
        def _drain():
            wait_out(cur)

        xb = x_buf[cur].astype(jnp.bfloat16)
        o_buf[cur] = (
            jnp.dot(xb, wb_ref[...], preferred_element_type=jnp.float32)
            + b_ref[...]
        )
        dma_out(cur, step)
        return ()

    jax.lax.fori_loop(0, n_steps, body, (), unroll=False)
    wait_out(jax.lax.rem(n_steps - 2, 2))
    wait_out(jax.lax.rem(n_steps - 1, 2))


def kernel(x, w, b):
    m, k = x.shape
    n = w.shape[1]
    cost = pl.CostEstimate(
        flops=2 * m * k * n,
        transcendentals=0,
        bytes_accessed=4 * m * k + 4 * k * n + 4 * n + 4 * m * n,
    )
    return pl.pallas_call(
        _linear_kernel,
        out_shape=jax.ShapeDtypeStruct((m, n), jnp.float32),
        in_specs=[
            pl.BlockSpec(memory_space=pltpu.MemorySpace.HBM),
            pl.BlockSpec(memory_space=pltpu.MemorySpace.VMEM),
            pl.BlockSpec(memory_space=pltpu.MemorySpace.VMEM),
        ],
        out_specs=pl.BlockSpec(memory_space=pltpu.MemorySpace.HBM),
        scratch_shapes=[
            pltpu.VMEM((k, n), jnp.bfloat16),
            pltpu.VMEM((2, _TM, k), jnp.float32),
            pltpu.VMEM((2, _TM, n), jnp.float32),
            pltpu.SemaphoreType.DMA((2,)),
            pltpu.SemaphoreType.DMA((2,)),
        ],
        compiler_params=pltpu.CompilerParams(
            vmem_limit_bytes=60 << 20,
        ),
        cost_estimate=cost,
    )(x, w, b)


# speedup vs baseline: 1.6406x; 1.0004x over previous
import jax
import jax.numpy as jnp
from jax.experimental import pallas as pl
from jax.experimental.pallas import tpu as pltpu

_TM = 512


def _linear_kernel(x_hbm, w_ref, b_ref, o_hbm, x_buf, o_buf, in_sem, out_sem):
    n_steps = x_hbm.shape[0] // _TM

    def dma_in(slot, step):
        pltpu.make_async_copy(
            x_hbm.at[pl.ds(step * _TM, _TM)], x_buf.at[slot], in_sem.at[slot]
        ).start()

    def wait_in(slot):
        pltpu.make_async_copy(
            x_hbm.at[pl.ds(0, _TM)], x_buf.at[slot], in_sem.at[slot]
        ).wait()

    def dma_out(slot, step):
        pltpu.make_async_copy(
            o_buf.at[slot], o_hbm.at[pl.ds(step * _TM, _TM)], out_sem.at[slot]
        ).start()

    def wait_out(slot):
        pltpu.make_async_copy(
            o_buf.at[slot], o_hbm.at[pl.ds(0, _TM)], out_sem.at[slot]
        ).wait()

    dma_in(0, 0)

    def body(step, _):
        cur = jax.lax.rem(step, 2)
        nxt = jax.lax.rem(step + 1, 2)

        @pl.when(step + 1 < n_steps)
        def _prefetch():
            dma_in(nxt, step + 1)

        wait_in(cur)

        @pl.when(step >= 2)
        def _drain():
            wait_out(cur)

        o_buf[cur] = (
            jnp.dot(x_buf[cur], w_ref[...], preferred_element_type=jnp.float32)
            + b_ref[...]
        )
        dma_out(cur, step)
        return ()

    jax.lax.fori_loop(0, n_steps, body, (), unroll=2)
    wait_out(jax.lax.rem(n_steps - 2, 2))
    wait_out(jax.lax.rem(n_steps - 1, 2))


def kernel(x, w, b):
    m, k = x.shape
    n = w.shape[1]
    cost = pl.CostEstimate(
        flops=2 * m * k * n,
        transcendentals=0,
        bytes_accessed=4 * m * k + 4 * k * n + 4 * n + 4 * m * n,
    )
    return pl.pallas_call(
        _linear_kernel,
        out_shape=jax.ShapeDtypeStruct((m, n), jnp.float32),
        in_specs=[
            pl.BlockSpec(memory_space=pltpu.MemorySpace.HBM),
            pl.BlockSpec(memory_space=pltpu.MemorySpace.VMEM),
            pl.BlockSpec(memory_space=pltpu.MemorySpace.VMEM),
        ],
        out_specs=pl.BlockSpec(memory_space=pltpu.MemorySpace.HBM),
        scratch_shapes=[
            pltpu.VMEM((2, _TM, k), jnp.float32),
            pltpu.VMEM((2, _TM, n), jnp.float32),
            pltpu.SemaphoreType.DMA((2,)),
            pltpu.SemaphoreType.DMA((2,)),
        ],
        compiler_params=pltpu.CompilerParams(
            vmem_limit_bytes=60 << 20,
        ),
        cost_estimate=cost,
    )(x, w, b)


# speedup vs baseline: 1.6428x; 1.0013x over previous
import jax
import jax.numpy as jnp
from jax.experimental import pallas as pl
from jax.experimental.pallas import tpu as pltpu

_TM = 512
_WSTRIPES = 4


def _linear_kernel(x_hbm, w_hbm, b_ref, o_hbm,
                   w_vmem, x_buf, o_buf, in_sem, out_sem, w_sem):
    n_steps = x_hbm.shape[0] // _TM
    wk = w_hbm.shape[0] // _WSTRIPES

    def w_stripe(q):
        return pltpu.make_async_copy(
            w_hbm.at[pl.ds(q * wk, wk)], w_vmem.at[pl.ds(q * wk, wk)],
            w_sem.at[q],
        )

    def dma_in(slot, step):
        pltpu.make_async_copy(
            x_hbm.at[pl.ds(step * _TM, _TM)], x_buf.at[slot], in_sem.at[slot]
        ).start()

    def wait_in(slot):
        pltpu.make_async_copy(
            x_hbm.at[pl.ds(0, _TM)], x_buf.at[slot], in_sem.at[slot]
        ).wait()

    def dma_out(slot, step):
        pltpu.make_async_copy(
            o_buf.at[slot], o_hbm.at[pl.ds(step * _TM, _TM)], out_sem.at[slot]
        ).start()

    def wait_out(slot):
        pltpu.make_async_copy(
            o_buf.at[slot], o_hbm.at[pl.ds(0, _TM)], out_sem.at[slot]
        ).wait()

    for q in range(_WSTRIPES):
        w_stripe(q).start()
    dma_in(0, 0)
    for q in range(_WSTRIPES):
        w_stripe(q).wait()

    def body(step, _):
        cur = jax.lax.rem(step, 2)
        nxt = jax.lax.rem(step + 1, 2)

        @pl.when(step + 1 < n_steps)
        def _prefetch():
            dma_in(nxt, step + 1)

        wait_in(cur)

        @pl.when(step >= 2)
        def _drain():
            wait_out(cur)

        o_buf[cur] = (
            jnp.dot(x_buf[cur], w_vmem[...], preferred_element_type=jnp.float32)
            + b_ref[...]
        )
        dma_out(cur, step)
        return ()

    jax.lax.fori_loop(0, n_steps, body, (), unroll=2)
    wait_out(jax.lax.rem(n_steps - 2, 2))
    wait_out(jax.lax.rem(n_steps - 1, 2))


def kernel(x, w, b):
    m, k = x.shape
    n = w.shape[1]
    cost = pl.CostEstimate(
        flops=2 * m * k * n,
        transcendentals=0,
        bytes_accessed=4 * m * k + 4 * k * n + 4 * n + 4 * m * n,
    )
    return pl.pallas_call(
        _linear_kernel,
        out_shape=jax.ShapeDtypeStruct((m, n), jnp.float32),
        in_specs=[
            pl.BlockSpec(memory_space=pltpu.MemorySpace.HBM),
            pl.BlockSpec(memory_space=pltpu.MemorySpace.HBM),
            pl.BlockSpec(memory_space=pltpu.MemorySpace.VMEM),
        ],
        out_specs=pl.BlockSpec(memory_space=pltpu.MemorySpace.HBM),
        scratch_shapes=[
            pltpu.VMEM((k, n), jnp.float32),
            pltpu.VMEM((2, _TM, k), jnp.float32),
            pltpu.VMEM((2, _TM, n), jnp.float32),
            pltpu.SemaphoreType.DMA((2,)),
            pltpu.SemaphoreType.DMA((2,)),
            pltpu.SemaphoreType.DMA((_WSTRIPES,)),
        ],
        compiler_params=pltpu.CompilerParams(
            vmem_limit_bytes=60 << 20,
        ),
        cost_estimate=cost,
    )(x, w, b)


# speedup vs baseline: 1.6433x; 1.0003x over previous
import jax
import jax.numpy as jnp
from jax.experimental import pallas as pl
from jax.experimental.pallas import tpu as pltpu

_TM = 512
_WSTRIPES = 4
_S = 2
_SM = _TM // _S
_NOB = 3


def _linear_kernel(x_hbm, w_hbm, b_ref, o_hbm,
                   w_vmem, x_buf, o_buf, in_sem, out_sem, w_sem):
    n_steps = x_hbm.shape[0] // _TM
    wk = w_hbm.shape[0] // _WSTRIPES

    def w_stripe(q):
        return pltpu.make_async_copy(
            w_hbm.at[pl.ds(q * wk, wk)], w_vmem.at[pl.ds(q * wk, wk)],
            w_sem.at[q],
        )

    def dma_in(slot, step):
        for h in range(_S):
            pltpu.make_async_copy(
                x_hbm.at[pl.ds(step * _TM + h * _SM, _SM)],
                x_buf.at[slot].at[pl.ds(h * _SM, _SM)],
                in_sem.at[slot, h],
            ).start()

    def wait_in(slot):
        for h in range(_S):
            pltpu.make_async_copy(
                x_hbm.at[pl.ds(0, _SM)],
                x_buf.at[slot].at[pl.ds(0, _SM)],
                in_sem.at[slot, h],
            ).wait()

    def dma_out(slot, step):
        for h in range(_S):
            pltpu.make_async_copy(
                o_buf.at[slot].at[pl.ds(h * _SM, _SM)],
                o_hbm.at[pl.ds(step * _TM + h * _SM, _SM)],
                out_sem.at[slot, h],
            ).start()

    def wait_out(slot):
        for h in range(_S):
            pltpu.make_async_copy(
                o_buf.at[slot].at[pl.ds(0, _SM)],
                o_hbm.at[pl.ds(0, _SM)],
                out_sem.at[slot, h],
            ).wait()

    for q in range(_WSTRIPES):
        w_stripe(q).start()
    dma_in(0, 0)
    for q in range(_WSTRIPES):
        w_stripe(q).wait()

    def body(step, _):
        cur = jax.lax.rem(step, 2)
        nxt = jax.lax.rem(step + 1, 2)
        ocur = jax.lax.rem(step, _NOB)

        @pl.when(step + 1 < n_steps)
        def _prefetch():
            dma_in(nxt, step + 1)

        wait_in(cur)

        @pl.when(step >= _NOB)
        def _drain():
            wait_out(ocur)

        o_buf[ocur] = (
            jnp.dot(x_buf[cur], w_vmem[...], preferred_element_type=jnp.float32)
            + b_ref[...]
        )
        dma_out(ocur, step)
        return ()

    jax.lax.fori_loop(0, n_steps, body, (), unroll=2)
    for t in range(_NOB):
        wait_out(jax.lax.rem(n_steps - _NOB + t, _NOB))


def kernel(x, w, b):
    m, k = x.shape
    n = w.shape[1]
    cost = pl.CostEstimate(
        flops=2 * m * k * n,
        transcendentals=0,
        bytes_accessed=4 * m * k + 4 * k * n + 4 * n + 4 * m * n,
    )
    return pl.pallas_call(
        _linear_kernel,
        out_shape=jax.ShapeDtypeStruct((m, n), jnp.float32),
        in_specs=[
            pl.BlockSpec(memory_space=pltpu.MemorySpace.HBM),
            pl.BlockSpec(memory_space=pltpu.MemorySpace.HBM),
            pl.BlockSpec(memory_space=pltpu.MemorySpace.VMEM),
        ],
        out_specs=pl.BlockSpec(memory_space=pltpu.MemorySpace.HBM),
        scratch_shapes=[
            pltpu.VMEM((k, n), jnp.float32),
            pltpu.VMEM((2, _TM, k), jnp.float32),
            pltpu.VMEM((_NOB, _TM, n), jnp.float32),
            pltpu.SemaphoreType.DMA((2, _S)),
            pltpu.SemaphoreType.DMA((_NOB, _S)),
            pltpu.SemaphoreType.DMA((_WSTRIPES,)),
        ],
        compiler_params=pltpu.CompilerParams(
            vmem_limit_bytes=60 << 20,
        ),
        cost_estimate=cost,
    )(x, w, b)


# speedup vs baseline: 1.6512x; 1.0048x over previous
import jax
import jax.numpy as jnp
from jax.experimental import pallas as pl
from jax.experimental.pallas import tpu as pltpu

_TM = 512
_WSTRIPES = 4
_S = 2
_SM = _TM // _S
_NXB = 3
_NOB = 3


def _linear_kernel(x_hbm, w_hbm, b_ref, o_hbm,
                   w_vmem, x_buf, o_buf, in_sem, out_sem, w_sem):
    n_steps = x_hbm.shape[0] // _TM
    wk = w_hbm.shape[0] // _WSTRIPES

    def w_stripe(q):
        return pltpu.make_async_copy(
            w_hbm.at[pl.ds(q * wk, wk)], w_vmem.at[pl.ds(q * wk, wk)],
            w_sem.at[q],
        )

    def dma_in(slot, step):
        for h in range(_S):
            pltpu.make_async_copy(
                x_hbm.at[pl.ds(step * _TM + h * _SM, _SM)],
                x_buf.at[slot].at[pl.ds(h * _SM, _SM)],
                in_sem.at[slot, h],
            ).start()

    def wait_in(slot):
        for h in range(_S):
            pltpu.make_async_copy(
                x_hbm.at[pl.ds(0, _SM)],
                x_buf.at[slot].at[pl.ds(0, _SM)],
                in_sem.at[slot, h],
            ).wait()

    def dma_out(slot, step):
        for h in range(_S):
            pltpu.make_async_copy(
                o_buf.at[slot].at[pl.ds(h * _SM, _SM)],
                o_hbm.at[pl.ds(step * _TM + h * _SM, _SM)],
                out_sem.at[slot, h],
            ).start()

    def wait_out(slot):
        for h in range(_S):
            pltpu.make_async_copy(
                o_buf.at[slot].at[pl.ds(0, _SM)],
                o_hbm.at[pl.ds(0, _SM)],
                out_sem.at[slot, h],
            ).wait()

    w_stripe(0).start()
    dma_in(0, 0)
    for q in range(1, _WSTRIPES):
        w_stripe(q).start()
    dma_in(1, 1)
    dma_in(2, 2)

    w_stripe(0).wait()
    wait_in(0)
    x0 = x_buf[0]
    o_buf[0] = b_ref[...] + jnp.dot(
        x0[:, 0:wk], w_vmem[0:wk, :], preferred_element_type=jnp.float32
    )
    for q in range(1, _WSTRIPES):
        w_stripe(q).wait()
        o_buf[0] += jnp.dot(
            x0[:, q * wk:(q + 1) * wk], w_vmem[q * wk:(q + 1) * wk, :],
            preferred_element_type=jnp.float32,
        )
    dma_out(0, 0)

    def body(step, _):
        cur = jax.lax.rem(step, _NXB)
        pre = jax.lax.rem(step + 2, _NXB)
        ocur = jax.lax.rem(step, _NOB)

        @pl.when(step + 2 < n_steps)
        def _prefetch():
            dma_in(pre, step + 2)

        wait_in(cur)

        @pl.when(step >= _NOB)
        def _drain():
            wait_out(ocur)

        o_buf[ocur] = (
            jnp.dot(x_buf[cur], w_vmem[...], preferred_element_type=jnp.float32)
            + b_ref[...]
        )
        dma_out(ocur, step)
        return ()

    jax.lax.fori_loop(1, n_steps, body, (), unroll=2)
    for t in range(n_steps - _NOB, n_steps):
        wait_out(jax.lax.rem(t, _NOB))


def kernel(x, w, b):
    m, k = x.shape
    n = w.shape[1]
    cost = pl.CostEstimate(
        flops=2 * m * k * n,
        transcendentals=0,
        bytes_accessed=4 * m * k + 4 * k * n + 4 * n + 4 * m * n,
    )
    return pl.pallas_call(
        _linear_kernel,
        out_shape=jax.ShapeDtypeStruct((m, n), jnp.float32),
        in_specs=[
            pl.BlockSpec(memory_space=pltpu.MemorySpace.HBM),
            pl.BlockSpec(memory_space=pltpu.MemorySpace.HBM),
            pl.BlockSpec(memory_space=pltpu.MemorySpace.VMEM),
        ],
        out_specs=pl.BlockSpec(memory_space=pltpu.MemorySpace.HBM),
        scratch_shapes=[
            pltpu.VMEM((k, n), jnp.float32),
            pltpu.VMEM((_NXB, _TM, k), jnp.float32),
            pltpu.VMEM((_NOB, _TM, n), jnp.float32),
            pltpu.SemaphoreType.DMA((_NXB, _S)),
            pltpu.SemaphoreType.DMA((_NOB, _S)),
            pltpu.SemaphoreType.DMA((_WSTRIPES,)),
        ],
        compiler_params=pltpu.CompilerParams(
            vmem_limit_bytes=60 << 20,
        ),
        cost_estimate=cost,
    )(x, w, b)
